# baseline (speedup 1.0000x reference)
import jax
import jax.numpy as jnp
from jax import lax
from jax.experimental import pallas as pl
from jax.experimental.pallas import tpu as pltpu
from jax.experimental.pallas import tpu_sc as plsc

B, T, V, D = 16384, 3, 10, 10
NC, NS = 2, 16
NW = NC * NS
BPW = B // NW
GROUPS = BPW // 16
CTS = 32


def _body(x_hbm, char_hbm, pos_hbm, out_hbm, x_v, char_v, pos_v, ct_v, out_v):
    wid = lax.axis_index("s") * NC + lax.axis_index("c")
    b0 = wid * BPW

    pltpu.sync_copy(x_hbm.at[pl.ds(b0 * T, BPW * T)], x_v)
    pltpu.sync_copy(char_hbm, char_v)
    pltpu.sync_copy(pos_hbm, pos_v)

    lane = lax.iota(jnp.int32, 16)
    zero_v = lane * 0

    for c0, width in ((0, 16), (16, T * V - 16)):
        c_vec = lane + c0
        mask_c = lane < width
        t_vec = ((c_vec >= V).astype(jnp.int32)
                 + (c_vec >= 2 * V).astype(jnp.int32))
        v10 = (c_vec - t_vec * V) * D
        t10 = t_vec * V
        for d in range(D):
            cv = plsc.load_gather(char_v, [v10 + d], mask=mask_c)
            pv = plsc.load_gather(pos_v, [t10 + d], mask=mask_c)
            plsc.store_scatter(ct_v, [c_vec + d * CTS], cv + pv, mask=mask_c)

    @plsc.parallel_loop(0, GROUPS, step=1, unroll=2)
    def _loop(i):
        lb = i * 16 + lane
        xb = lb * T
        ob = lb * CTS
        for t in range(T):
            xv = plsc.load_gather(x_v, [xb + t])
            cidx = xv + t * V
            for d in range(D):
                val = plsc.load_gather(ct_v, [cidx + d * CTS])
                plsc.store_scatter(out_v, [zero_v, ob + (t * D + d)], val)

    pltpu.sync_copy(out_v, out_hbm.at[pl.ds(b0, BPW)])


def kernel(x, char_table, pos_table):
    mesh = plsc.VectorSubcoreMesh(
        core_axis_name="c", subcore_axis_name="s",
        num_cores=NC, num_subcores=NS)
    k = pl.kernel(
        _body,
        out_type=jax.ShapeDtypeStruct((B, T * D), jnp.float32),
        mesh=mesh,
        scratch_types=[
            pltpu.VMEM((BPW * T,), jnp.int32),
            pltpu.VMEM((V * D,), jnp.float32),
            pltpu.VMEM((T * V,), jnp.float32),
            pltpu.VMEM((D * CTS,), jnp.float32),
            pltpu.VMEM((BPW, T * D), jnp.float32),
        ],
        compiler_params=pltpu.CompilerParams(
            needs_layout_passes=False, use_tc_tiling_on_sc=False),
    )
    out = k(x.reshape(B * T), char_table.reshape(V * D),
            pos_table.reshape(T * V))
    return out.reshape(B, T, D)


# speedup vs baseline: 1.2372x; 1.2372x over previous
import jax
import jax.numpy as jnp
from jax import lax
from jax.experimental import pallas as pl
from jax.experimental.pallas import tpu as pltpu
from jax.experimental.pallas import tpu_sc as plsc

B, T, V, D = 16384, 3, 10, 10
NC, NS = 2, 16
NW = NC * NS
BPW = B // NW
GROUPS = BPW // 16
CTS = 32


def _body(x_hbm, char_hbm, pos_hbm, out_hbm, x_v, char_v, pos_v, ct_v, out_v):
    wid = lax.axis_index("s") * NC + lax.axis_index("c")
    b0 = wid * BPW

    for t in range(T):
        pltpu.sync_copy(x_hbm.at[pl.ds(t * B + b0, BPW)],
                        x_v.at[pl.ds(t * BPW, BPW)])
    pltpu.sync_copy(char_hbm, char_v)
    pltpu.sync_copy(pos_hbm, pos_v)

    lane = lax.iota(jnp.int32, 16)
    zero_v = lane * 0

    for c0, width in ((0, 16), (16, T * V - 16)):
        c_vec = lane + c0
        mask_c = lane < width
        t_vec = ((c_vec >= V).astype(jnp.int32)
                 + (c_vec >= 2 * V).astype(jnp.int32))
        v10 = (c_vec - t_vec * V) * D
        t10 = t_vec * V
        for d in range(D):
            cv = plsc.load_gather(char_v, [v10 + d], mask=mask_c)
            pv = plsc.load_gather(pos_v, [t10 + d], mask=mask_c)
            plsc.store_scatter(ct_v, [c_vec + d * CTS], cv + pv, mask=mask_c)

    @plsc.parallel_loop(0, GROUPS, step=1, unroll=2)
    def _loop(i):
        lb = i * 16 + lane
        ob = lb * CTS
        for t in range(T):
            xv = plsc.load_gather(x_v, [lb + t * BPW])
            cidx = xv + t * V
            for d in range(D):
                val = plsc.load_gather(ct_v, [cidx + d * CTS])
                plsc.store_scatter(out_v, [zero_v, ob + (t * D + d)], val)

    pltpu.sync_copy(out_v, out_hbm.at[pl.ds(b0, BPW)])


def kernel(x, char_table, pos_table):
    mesh = plsc.VectorSubcoreMesh(
        core_axis_name="c", subcore_axis_name="s",
        num_cores=NC, num_subcores=NS)
    k = pl.kernel(
        _body,
        out_type=jax.ShapeDtypeStruct((B, T * D), jnp.float32),
        mesh=mesh,
        scratch_types=[
            pltpu.VMEM((BPW * T,), jnp.int32),
            pltpu.VMEM((V * D,), jnp.float32),
            pltpu.VMEM((T * V,), jnp.float32),
            pltpu.VMEM((D * CTS,), jnp.float32),
            pltpu.VMEM((BPW, T * D), jnp.float32),
        ],
        compiler_params=pltpu.CompilerParams(
            needs_layout_passes=False, use_tc_tiling_on_sc=False),
    )
    out = k(x.T.reshape(T * B), char_table.reshape(V * D),
            pos_table.reshape(T * V))
    return out.reshape(B, T, D)


# speedup vs baseline: 1.8940x; 1.5310x over previous
import jax
import jax.numpy as jnp
from jax import lax
from jax.experimental import pallas as pl
from jax.experimental.pallas import tpu as pltpu
from jax.experimental.pallas import tpu_sc as plsc

B, T, V, D = 16384, 3, 10, 10
NC, NS = 2, 16
NW = NC * NS
BPW = B // NW
GROUPS = BPW // 16
CTS = 32


def _body(x_hbm, char_hbm, pos_hbm, out_hbm, x_v, char_v, pos_v, ct_v, out_v):
    wid = lax.axis_index("s") * NC + lax.axis_index("c")
    b0 = wid * BPW

    for t in range(T):
        pltpu.sync_copy(x_hbm.at[pl.ds(t * B + b0, BPW)],
                        x_v.at[pl.ds(t * BPW, BPW)])
    pltpu.sync_copy(char_hbm, char_v)
    pltpu.sync_copy(pos_hbm, pos_v)

    lane = lax.iota(jnp.int32, 16)
    zero_v = lane * 0

    for c0, width in ((0, 16), (16, T * V - 16)):
        c_vec = lane + c0
        mask_c = lane < width
        t_vec = ((c_vec >= V).astype(jnp.int32)
                 + (c_vec >= 2 * V).astype(jnp.int32))
        v10 = (c_vec - t_vec * V) * D
        t10 = t_vec * V
        for d in range(D):
            cv = plsc.load_gather(char_v, [v10 + d], mask=mask_c)
            pv = plsc.load_gather(pos_v, [t10 + d], mask=mask_c)
            plsc.store_scatter(ct_v, [c_vec + d * CTS], cv + pv, mask=mask_c)

    @plsc.parallel_loop(0, GROUPS, step=1, unroll=2)
    def _loop(i):
        lb = i * 16 + lane
        for t in range(T):
            xv = plsc.load_gather(x_v, [lb + t * BPW])
            cidx = xv + t * V
            for d in range(D):
                val = plsc.load_gather(ct_v, [cidx + d * CTS])
                plsc.store_scatter(out_v,
                                   [zero_v, (t * D + d) * BPW + lb], val)

    pltpu.sync_copy(out_v, out_hbm.at[:, pl.ds(b0, BPW)])


def kernel(x, char_table, pos_table):
    mesh = plsc.VectorSubcoreMesh(
        core_axis_name="c", subcore_axis_name="s",
        num_cores=NC, num_subcores=NS)
    k = pl.kernel(
        _body,
        out_type=jax.ShapeDtypeStruct((T * D, B), jnp.float32),
        mesh=mesh,
        scratch_types=[
            pltpu.VMEM((BPW * T,), jnp.int32),
            pltpu.VMEM((V * D,), jnp.float32),
            pltpu.VMEM((T * V,), jnp.float32),
            pltpu.VMEM((D * CTS,), jnp.float32),
            pltpu.VMEM((T * D, BPW), jnp.float32),
        ],
        compiler_params=pltpu.CompilerParams(
            needs_layout_passes=False, use_tc_tiling_on_sc=False),
    )
    out = k(x.T.reshape(T * B), char_table.reshape(V * D),
            pos_table.reshape(T * V))
    return out.T.reshape(B, T, D)
